# baseline (speedup 1.0000x reference)
import jax
import jax.numpy as jnp
from jax.experimental import pallas as pl


def _pos_broadcast_kernel(row_ref, col_ref, out_ref):
    half, H, W = out_ref.shape[1] // 2, out_ref.shape[2], out_ref.shape[3]
    cols_t = col_ref[...].T
    rows_t = row_ref[...].T
    out_ref[0, :half] = jnp.broadcast_to(cols_t[:, None, :], (half, H, W))
    out_ref[0, half:] = jnp.broadcast_to(rows_t[:, :, None], (half, H, W))


def kernel(feature, row_embed, col_embed):
    B, C, H, W = feature.shape
    half = C // 2
    return pl.pallas_call(
        _pos_broadcast_kernel,
        grid=(B,),
        in_specs=[
            pl.BlockSpec((H, half), lambda b: (0, 0)),
            pl.BlockSpec((W, half), lambda b: (0, 0)),
        ],
        out_specs=pl.BlockSpec((1, C, H, W), lambda b: (b, 0, 0, 0)),
        out_shape=jax.ShapeDtypeStruct((B, C, H, W), row_embed.dtype),
    )(row_embed, col_embed)


# speedup vs baseline: 1.5891x; 1.5891x over previous
import jax
import jax.numpy as jnp
from jax.experimental import pallas as pl
from jax.experimental.pallas import tpu as pltpu


def _pos_broadcast_kernel(row_ref, col_ref, out_ref):
    half = out_ref.shape[1] // 2
    H = row_ref.shape[0]
    W = col_ref.shape[0]
    cols_t = col_ref[...].T
    rows_t = row_ref[...].T
    out_ref[0, :half, :] = pltpu.repeat(cols_t, H, axis=1)
    out_ref[0, half:, :] = jnp.repeat(rows_t, W, axis=1)


def kernel(feature, row_embed, col_embed):
    B, C, H, W = feature.shape
    half = C // 2
    out = pl.pallas_call(
        _pos_broadcast_kernel,
        grid=(B,),
        in_specs=[
            pl.BlockSpec((H, half), lambda b: (0, 0)),
            pl.BlockSpec((W, half), lambda b: (0, 0)),
        ],
        out_specs=pl.BlockSpec((1, C, H * W), lambda b: (b, 0, 0)),
        out_shape=jax.ShapeDtypeStruct((B, C, H * W), row_embed.dtype),
    )(row_embed, col_embed)
    return out.reshape(B, C, H, W)


# speedup vs baseline: 1.7475x; 1.0996x over previous
import jax
import jax.numpy as jnp
from jax.experimental import pallas as pl
from jax.experimental.pallas import tpu as pltpu


def _make_pos_broadcast_kernel(B, H, W, half):
    def _pos_broadcast_kernel(row_ref, col_ref, out_ref, scratch, sem):
        cols_t = col_ref[:W, :].T
        rows_t = row_ref[:H, :].T
        scratch[:half, :] = pltpu.repeat(cols_t, H, axis=1)
        scratch[half:, :] = jnp.repeat(rows_t, W, axis=1)
        for b in range(B):
            pltpu.make_async_copy(scratch, out_ref.at[b], sem).start()
        for b in range(B):
            pltpu.make_async_copy(scratch, out_ref.at[b], sem).wait()

    return _pos_broadcast_kernel


def kernel(feature, row_embed, col_embed):
    B, C, H, W = feature.shape
    half = C // 2
    out = pl.pallas_call(
        _make_pos_broadcast_kernel(B, H, W, half),
        in_specs=[
            pl.BlockSpec(memory_space=pltpu.MemorySpace.VMEM),
            pl.BlockSpec(memory_space=pltpu.MemorySpace.VMEM),
        ],
        out_specs=pl.BlockSpec(memory_space=pl.ANY),
        out_shape=jax.ShapeDtypeStruct((B, C, H * W), row_embed.dtype),
        scratch_shapes=[
            pltpu.VMEM((C, H * W), row_embed.dtype),
            pltpu.SemaphoreType.DMA,
        ],
    )(row_embed, col_embed)
    return out.reshape(B, C, H, W)
